# Initial kernel scaffold; baseline (speedup 1.0000x reference)
import jax
import jax.numpy as jnp
from jax.experimental import pallas as pl


def kernel(x, edge_index, W1, b1, W2, b2):
    raise NotImplementedError("write your pallas kernel here")


# speedup vs baseline: 20.7259x; 20.7259x over previous
import functools

import jax
import jax.numpy as jnp
from jax import lax
from jax.experimental import pallas as pl
from jax.experimental.pallas import tpu as pltpu
from jax.experimental.pallas import tpu_sc as plsc

N = 10000
E = 320000
D = 128
H = 64
C = 40
C_PAD = 48

N_PAD = 10240
ROWS_PER_TILE = 640
NW = 32
CHUNK = 128
NCHUNK = 81
E_PAD = NW * NCHUNK * CHUNK

RB = 256
GRID = N_PAD // RB

_MESH = plsc.VectorSubcoreMesh(core_axis_name="c", subcore_axis_name="s")
_SC_PARAMS = pltpu.CompilerParams(use_tc_tiling_on_sc=False)


def _fill(buf, rows, width, value):
    def body(i, _):
        for k in range(width // 16):
            buf[i, pl.ds(16 * k, 16)] = jnp.full((16,), value, jnp.float32)
        return 0
    lax.fori_loop(0, rows, body, 0)


@functools.partial(
    pl.kernel,
    out_type=jax.ShapeDtypeStruct((2, N_PAD, 16), jnp.float32),
    mesh=_MESH,
    compiler_params=_SC_PARAMS,
    scratch_types=[
        pltpu.VMEM((NCHUNK, CHUNK), jnp.int32),
        pltpu.VMEM((CHUNK, 16), jnp.float32),
        pltpu.VMEM((CHUNK, 16), jnp.float32),
        pltpu.VMEM_SHARED((N_PAD, 16), jnp.float32),
    ],
)
def _sc_degree(dst_hbm, out_hbm, dst_v, ones_v, zbuf_v, acc):
    c = lax.axis_index("c")
    s = lax.axis_index("s")
    w = s * 2 + c
    _fill(ones_v, CHUNK, 16, 1.0)
    _fill(zbuf_v, CHUNK, 16, 0.0)
    base = s * ROWS_PER_TILE
    for t in range(ROWS_PER_TILE // CHUNK):
        pltpu.sync_copy(zbuf_v, acc.at[pl.ds(base + t * CHUNK, CHUNK)])
    plsc.subcore_barrier()
    pltpu.sync_copy(dst_hbm.at[w], dst_v)

    def step(j, _):
        pltpu.sync_copy(ones_v, acc.at[dst_v.at[j]], add=True)
        return 0
    lax.fori_loop(0, NCHUNK, step, 0)
    plsc.subcore_barrier()
    for t in range(ROWS_PER_TILE // CHUNK):
        pltpu.sync_copy(acc.at[pl.ds(base + t * CHUNK, CHUNK)], zbuf_v)
        pltpu.sync_copy(zbuf_v, out_hbm.at[c, pl.ds(base + t * CHUNK, CHUNK)])


def _make_sc_agg(feat):
    @functools.partial(
        pl.kernel,
        out_type=jax.ShapeDtypeStruct((2, N_PAD, feat), jnp.float32),
        mesh=_MESH,
        compiler_params=_SC_PARAMS,
        scratch_types=[
            pltpu.VMEM((NCHUNK, CHUNK), jnp.int32),
            pltpu.VMEM((NCHUNK, CHUNK), jnp.int32),
            pltpu.VMEM((CHUNK, feat), jnp.float32),
            pltpu.VMEM_SHARED((N_PAD, feat), jnp.float32),
            pltpu.SemaphoreType.DMA,
        ],
    )
    def agg(hs_hbm, src_hbm, dst_hbm, out_hbm, src_v, dst_v, buf, acc, sem):
        c = lax.axis_index("c")
        s = lax.axis_index("s")
        w = s * 2 + c
        _fill(buf, CHUNK, feat, 0.0)
        base = s * ROWS_PER_TILE
        for t in range(ROWS_PER_TILE // CHUNK):
            pltpu.sync_copy(buf, acc.at[pl.ds(base + t * CHUNK, CHUNK)])
        plsc.subcore_barrier()
        pltpu.sync_copy(src_hbm.at[w], src_v)
        pltpu.sync_copy(dst_hbm.at[w], dst_v)

        def step(j, _):
            pltpu.async_copy(hs_hbm.at[src_v.at[j]], buf, sem).wait()
            pltpu.sync_copy(buf, acc.at[dst_v.at[j]], add=True)
            return 0
        lax.fori_loop(0, NCHUNK, step, 0)
        plsc.subcore_barrier()
        for t in range(ROWS_PER_TILE // CHUNK):
            pltpu.sync_copy(acc.at[pl.ds(base + t * CHUNK, CHUNK)], buf)
            pltpu.sync_copy(buf, out_hbm.at[c, pl.ds(base + t * CHUNK, CHUNK)])
    return agg


_sc_agg64 = _make_sc_agg(H)
_sc_agg48 = _make_sc_agg(C_PAD)


def _tc1_body(deg_ref, x_ref, w1_ref, hs1_ref, dinv_ref):
    degs = deg_ref[0] + deg_ref[1]
    deg = degs[:, 0:1]
    dinv = jnp.where(deg > 0, lax.rsqrt(deg), 0.0)
    h1 = jnp.dot(x_ref[...], w1_ref[...], preferred_element_type=jnp.float32)
    hs1_ref[...] = h1 * dinv
    dinv_ref[...] = dinv


def _tc2_body(p_ref, dinv_ref, b1_ref, w2_ref, hs2_ref):
    dinv = dinv_ref[...]
    z = (p_ref[0] + p_ref[1]) * dinv + b1_ref[...]
    r = jnp.maximum(z, 0.0)
    h2 = jnp.dot(r, w2_ref[...], preferred_element_type=jnp.float32)
    hs2_ref[...] = h2 * dinv


def _tc3_body(p_ref, dinv_ref, b2_ref, lp_ref, z_ref):
    z = (p_ref[0] + p_ref[1]) * dinv_ref[...] + b2_ref[...]
    col = lax.broadcasted_iota(jnp.int32, (RB, C_PAD), 1)
    valid = col < C
    zm = jnp.where(valid, z, -jnp.inf)
    m = jnp.max(zm, axis=1, keepdims=True)
    e = jnp.where(valid, jnp.exp(z - m), 0.0)
    ssum = jnp.sum(e, axis=1, keepdims=True)
    lp_ref[...] = z - m - jnp.log(ssum)
    z_ref[...] = z


_tc1 = pl.pallas_call(
    _tc1_body,
    grid=(GRID,),
    in_specs=[
        pl.BlockSpec((2, RB, 16), lambda i: (0, i, 0)),
        pl.BlockSpec((RB, D), lambda i: (i, 0)),
        pl.BlockSpec((D, H), lambda i: (0, 0)),
    ],
    out_specs=[
        pl.BlockSpec((RB, H), lambda i: (i, 0)),
        pl.BlockSpec((RB, 1), lambda i: (i, 0)),
    ],
    out_shape=[
        jax.ShapeDtypeStruct((N_PAD, H), jnp.float32),
        jax.ShapeDtypeStruct((N_PAD, 1), jnp.float32),
    ],
)

_tc2 = pl.pallas_call(
    _tc2_body,
    grid=(GRID,),
    in_specs=[
        pl.BlockSpec((2, RB, H), lambda i: (0, i, 0)),
        pl.BlockSpec((RB, 1), lambda i: (i, 0)),
        pl.BlockSpec((1, H), lambda i: (0, 0)),
        pl.BlockSpec((H, C_PAD), lambda i: (0, 0)),
    ],
    out_specs=pl.BlockSpec((RB, C_PAD), lambda i: (i, 0)),
    out_shape=jax.ShapeDtypeStruct((N_PAD, C_PAD), jnp.float32),
)

_tc3 = pl.pallas_call(
    _tc3_body,
    grid=(GRID,),
    in_specs=[
        pl.BlockSpec((2, RB, C_PAD), lambda i: (0, i, 0)),
        pl.BlockSpec((RB, 1), lambda i: (i, 0)),
        pl.BlockSpec((1, C_PAD), lambda i: (0, 0)),
    ],
    out_specs=[
        pl.BlockSpec((RB, C_PAD), lambda i: (i, 0)),
        pl.BlockSpec((RB, C_PAD), lambda i: (i, 0)),
    ],
    out_shape=[
        jax.ShapeDtypeStruct((N_PAD, C_PAD), jnp.float32),
        jax.ShapeDtypeStruct((N_PAD, C_PAD), jnp.float32),
    ],
)


def kernel(x, edge_index, W1, b1, W2, b2):
    loop = jnp.arange(N, dtype=jnp.int32)
    fill = jnp.full((E_PAD - E - N,), N, dtype=jnp.int32)
    src = jnp.concatenate([edge_index[0], loop, fill]).reshape(NW, NCHUNK, CHUNK)
    dst = jnp.concatenate([edge_index[1], loop, fill]).reshape(NW, NCHUNK, CHUNK)

    xp = jnp.pad(x, ((0, N_PAD - N), (0, 0)))
    w2p = jnp.pad(W2, ((0, 0), (0, C_PAD - C)))
    b1r = b1.reshape(1, H)
    b2r = jnp.pad(b2, (0, C_PAD - C)).reshape(1, C_PAD)

    deg_parts = _sc_degree(dst)
    hs1, dinv = _tc1(deg_parts, xp, W1)
    agg1 = _sc_agg64(hs1, src, dst)
    hs2 = _tc2(agg1, dinv, b1r, w2p)
    agg2 = _sc_agg48(hs2, src, dst)
    lp, z = _tc3(agg2, dinv, b2r)

    return (lp[:N, :C], z[:N, :C], jnp.float32(0.0))


# speedup vs baseline: 25.9116x; 1.2502x over previous
import functools

import jax
import jax.numpy as jnp
from jax import lax
from jax.experimental import pallas as pl
from jax.experimental.pallas import tpu as pltpu
from jax.experimental.pallas import tpu_sc as plsc

N = 10000
E = 320000
D = 128
H = 64
C = 40
C_PAD = 48

N_PAD = 10240
ROWS_PER_TILE = 640
NW = 32
CHUNK = 128
NCHUNK = 81
E_PAD = NW * NCHUNK * CHUNK

RB = 256
GRID = N_PAD // RB

_MESH = plsc.VectorSubcoreMesh(core_axis_name="c", subcore_axis_name="s")
_SC_PARAMS = pltpu.CompilerParams(use_tc_tiling_on_sc=False)


def _fill(buf, rows, width, value):
    def body(i, _):
        for k in range(width // 16):
            buf[i, pl.ds(16 * k, 16)] = jnp.full((16,), value, jnp.float32)
        return 0
    lax.fori_loop(0, rows, body, 0)


@functools.partial(
    pl.kernel,
    out_type=jax.ShapeDtypeStruct((2, N_PAD, 16), jnp.float32),
    mesh=_MESH,
    compiler_params=_SC_PARAMS,
    scratch_types=[
        pltpu.VMEM((NCHUNK, CHUNK), jnp.int32),
        pltpu.VMEM((CHUNK, 16), jnp.float32),
        pltpu.VMEM((CHUNK, 16), jnp.float32),
        pltpu.VMEM_SHARED((N_PAD, 16), jnp.float32),
        pltpu.SemaphoreType.DMA,
    ],
)
def _sc_degree(dst_hbm, out_hbm, dst_v, ones_v, zbuf_v, acc, sem):
    c = lax.axis_index("c")
    s = lax.axis_index("s")
    w = s * 2 + c
    _fill(ones_v, CHUNK, 16, 1.0)
    _fill(zbuf_v, CHUNK, 16, 0.0)
    base = s * ROWS_PER_TILE
    for t in range(ROWS_PER_TILE // CHUNK):
        pltpu.sync_copy(zbuf_v, acc.at[pl.ds(base + t * CHUNK, CHUNK)])
    plsc.subcore_barrier()
    pltpu.sync_copy(dst_hbm.at[w], dst_v)

    def step(j, _):
        pltpu.async_copy(ones_v, acc.at[dst_v.at[j]], sem, add=True)
        return 0
    lax.fori_loop(0, NCHUNK, step, 0)

    def drain(j, _):
        pltpu.make_async_copy(ones_v, acc.at[dst_v.at[j]], sem).wait()
        return 0
    lax.fori_loop(0, NCHUNK, drain, 0)
    plsc.subcore_barrier()
    for t in range(ROWS_PER_TILE // CHUNK):
        pltpu.sync_copy(acc.at[pl.ds(base + t * CHUNK, CHUNK)], zbuf_v)
        pltpu.sync_copy(zbuf_v, out_hbm.at[c, pl.ds(base + t * CHUNK, CHUNK)])


def _make_sc_agg(feat):
    @functools.partial(
        pl.kernel,
        out_type=jax.ShapeDtypeStruct((2, N_PAD, feat), jnp.float32),
        mesh=_MESH,
        compiler_params=_SC_PARAMS,
        scratch_types=[
            pltpu.VMEM((NCHUNK, CHUNK), jnp.int32),
            pltpu.VMEM((NCHUNK, CHUNK), jnp.int32),
            pltpu.VMEM((CHUNK, feat), jnp.float32),
            pltpu.VMEM((CHUNK, feat), jnp.float32),
            pltpu.VMEM_SHARED((N_PAD, feat), jnp.float32),
            pltpu.SemaphoreType.DMA,
            pltpu.SemaphoreType.DMA,
            pltpu.SemaphoreType.DMA,
            pltpu.SemaphoreType.DMA,
        ],
    )
    def agg(hs_hbm, src_hbm, dst_hbm, out_hbm,
            src_v, dst_v, buf0, buf1, acc, gsem0, gsem1, ssem0, ssem1):
        c = lax.axis_index("c")
        s = lax.axis_index("s")
        w = s * 2 + c
        _fill(buf0, CHUNK, feat, 0.0)
        base = s * ROWS_PER_TILE
        for t in range(ROWS_PER_TILE // CHUNK):
            pltpu.sync_copy(buf0, acc.at[pl.ds(base + t * CHUNK, CHUNK)])
        plsc.subcore_barrier()
        pltpu.sync_copy(src_hbm.at[w], src_v)
        pltpu.sync_copy(dst_hbm.at[w], dst_v)

        def gather(j, buf, sem):
            pltpu.async_copy(hs_hbm.at[src_v.at[j]], buf, sem)

        def gwait(j, buf, sem):
            pltpu.make_async_copy(hs_hbm.at[src_v.at[j]], buf, sem).wait()

        def scatter(j, buf, sem):
            pltpu.async_copy(buf, acc.at[dst_v.at[j]], sem, add=True)

        def swait(j, buf, sem):
            pltpu.make_async_copy(buf, acc.at[dst_v.at[j]], sem).wait()

        gather(0, buf0, gsem0)
        gather(1, buf1, gsem1)
        gwait(0, buf0, gsem0)
        scatter(0, buf0, ssem0)

        def pair(t, _):
            j = 2 * t
            swait(j - 2, buf0, ssem0)
            gather(j, buf0, gsem0)
            gwait(j - 1, buf1, gsem1)
            scatter(j - 1, buf1, ssem1)
            swait(j - 1, buf1, ssem1)
            gather(j + 1, buf1, gsem1)
            gwait(j, buf0, gsem0)
            scatter(j, buf0, ssem0)
            return 0
        lax.fori_loop(1, (NCHUNK - 1) // 2, pair, 0)

        j = NCHUNK - 1
        swait(j - 2, buf0, ssem0)
        gather(j, buf0, gsem0)
        gwait(j - 1, buf1, gsem1)
        scatter(j - 1, buf1, ssem1)
        gwait(j, buf0, gsem0)
        scatter(j, buf0, ssem0)
        swait(j - 1, buf1, ssem1)
        swait(j, buf0, ssem0)
        plsc.subcore_barrier()
        for t in range(ROWS_PER_TILE // CHUNK):
            pltpu.sync_copy(acc.at[pl.ds(base + t * CHUNK, CHUNK)], buf0)
            pltpu.sync_copy(buf0, out_hbm.at[c, pl.ds(base + t * CHUNK, CHUNK)])
    return agg


_sc_agg64 = _make_sc_agg(H)
_sc_agg48 = _make_sc_agg(C_PAD)


def _tc1_body(deg_ref, x_ref, w1_ref, hs1_ref, dinv_ref):
    degs = deg_ref[0] + deg_ref[1]
    deg = degs[:, 0:1]
    dinv = jnp.where(deg > 0, lax.rsqrt(deg), 0.0)
    h1 = jnp.dot(x_ref[...], w1_ref[...], preferred_element_type=jnp.float32)
    hs1_ref[...] = h1 * dinv
    dinv_ref[...] = dinv


def _tc2_body(p_ref, dinv_ref, b1_ref, w2_ref, hs2_ref):
    dinv = dinv_ref[...]
    z = (p_ref[0] + p_ref[1]) * dinv + b1_ref[...]
    r = jnp.maximum(z, 0.0)
    h2 = jnp.dot(r, w2_ref[...], preferred_element_type=jnp.float32)
    hs2_ref[...] = h2 * dinv


def _tc3_body(p_ref, dinv_ref, b2_ref, lp_ref, z_ref):
    z = (p_ref[0] + p_ref[1]) * dinv_ref[...] + b2_ref[...]
    col = lax.broadcasted_iota(jnp.int32, (RB, C_PAD), 1)
    valid = col < C
    zm = jnp.where(valid, z, -jnp.inf)
    m = jnp.max(zm, axis=1, keepdims=True)
    e = jnp.where(valid, jnp.exp(z - m), 0.0)
    ssum = jnp.sum(e, axis=1, keepdims=True)
    lp_ref[...] = z - m - jnp.log(ssum)
    z_ref[...] = z


_tc1 = pl.pallas_call(
    _tc1_body,
    grid=(GRID,),
    in_specs=[
        pl.BlockSpec((2, RB, 16), lambda i: (0, i, 0)),
        pl.BlockSpec((RB, D), lambda i: (i, 0)),
        pl.BlockSpec((D, H), lambda i: (0, 0)),
    ],
    out_specs=[
        pl.BlockSpec((RB, H), lambda i: (i, 0)),
        pl.BlockSpec((RB, 1), lambda i: (i, 0)),
    ],
    out_shape=[
        jax.ShapeDtypeStruct((N_PAD, H), jnp.float32),
        jax.ShapeDtypeStruct((N_PAD, 1), jnp.float32),
    ],
)

_tc2 = pl.pallas_call(
    _tc2_body,
    grid=(GRID,),
    in_specs=[
        pl.BlockSpec((2, RB, H), lambda i: (0, i, 0)),
        pl.BlockSpec((RB, 1), lambda i: (i, 0)),
        pl.BlockSpec((1, H), lambda i: (0, 0)),
        pl.BlockSpec((H, C_PAD), lambda i: (0, 0)),
    ],
    out_specs=pl.BlockSpec((RB, C_PAD), lambda i: (i, 0)),
    out_shape=jax.ShapeDtypeStruct((N_PAD, C_PAD), jnp.float32),
)

_tc3 = pl.pallas_call(
    _tc3_body,
    grid=(GRID,),
    in_specs=[
        pl.BlockSpec((2, RB, C_PAD), lambda i: (0, i, 0)),
        pl.BlockSpec((RB, 1), lambda i: (i, 0)),
        pl.BlockSpec((1, C_PAD), lambda i: (0, 0)),
    ],
    out_specs=[
        pl.BlockSpec((RB, C_PAD), lambda i: (i, 0)),
        pl.BlockSpec((RB, C_PAD), lambda i: (i, 0)),
    ],
    out_shape=[
        jax.ShapeDtypeStruct((N_PAD, C_PAD), jnp.float32),
        jax.ShapeDtypeStruct((N_PAD, C_PAD), jnp.float32),
    ],
)


def kernel(x, edge_index, W1, b1, W2, b2):
    loop = jnp.arange(N, dtype=jnp.int32)
    fill = jnp.full((E_PAD - E - N,), N, dtype=jnp.int32)
    src = jnp.concatenate([edge_index[0], loop, fill]).reshape(NW, NCHUNK, CHUNK)
    dst = jnp.concatenate([edge_index[1], loop, fill]).reshape(NW, NCHUNK, CHUNK)

    xp = jnp.pad(x, ((0, N_PAD - N), (0, 0)))
    w2p = jnp.pad(W2, ((0, 0), (0, C_PAD - C)))
    b1r = b1.reshape(1, H)
    b2r = jnp.pad(b2, (0, C_PAD - C)).reshape(1, C_PAD)

    deg_parts = _sc_degree(dst)
    hs1, dinv = _tc1(deg_parts, xp, W1)
    agg1 = _sc_agg64(hs1, src, dst)
    hs2 = _tc2(agg1, dinv, b1r, w2p)
    agg2 = _sc_agg48(hs2, src, dst)
    lp, z = _tc3(agg2, dinv, b2r)

    return (lp[:N, :C], z[:N, :C], jnp.float32(0.0))


# speedup vs baseline: 27.3578x; 1.0558x over previous
import functools

import jax
import jax.numpy as jnp
from jax import lax
from jax.experimental import pallas as pl
from jax.experimental.pallas import tpu as pltpu
from jax.experimental.pallas import tpu_sc as plsc

N = 10000
E = 320000
D = 128
H = 64
C = 40
C_PAD = 48

N_PAD = 10240
ROWS_PER_TILE = 640
NW = 32
CHUNK = 128
NCHUNK = 81
RING = 4
E_PAD = NW * NCHUNK * CHUNK

RB = 256
GRID = N_PAD // RB

_MESH = plsc.VectorSubcoreMesh(core_axis_name="c", subcore_axis_name="s")
_SC_PARAMS = pltpu.CompilerParams(use_tc_tiling_on_sc=False)


def _fill(buf, rows, width, value):
    def body(i, _):
        for k in range(width // 16):
            buf[i, pl.ds(16 * k, 16)] = jnp.full((16,), value, jnp.float32)
        return 0
    lax.fori_loop(0, rows, body, 0)


@functools.partial(
    pl.kernel,
    out_type=jax.ShapeDtypeStruct((2, N_PAD, 16), jnp.float32),
    mesh=_MESH,
    compiler_params=_SC_PARAMS,
    scratch_types=[
        pltpu.VMEM((NCHUNK, CHUNK), jnp.int32),
        pltpu.VMEM((CHUNK, 16), jnp.float32),
        pltpu.VMEM((CHUNK, 16), jnp.float32),
        pltpu.VMEM_SHARED((N_PAD, 16), jnp.float32),
        pltpu.SemaphoreType.DMA,
    ],
)
def _sc_degree(dst_hbm, out_hbm, dst_v, ones_v, zbuf_v, acc, sem):
    c = lax.axis_index("c")
    s = lax.axis_index("s")
    w = s * 2 + c
    _fill(ones_v, CHUNK, 16, 1.0)
    _fill(zbuf_v, CHUNK, 16, 0.0)
    base = s * ROWS_PER_TILE
    for t in range(ROWS_PER_TILE // CHUNK):
        pltpu.sync_copy(zbuf_v, acc.at[pl.ds(base + t * CHUNK, CHUNK)])
    plsc.subcore_barrier()
    pltpu.sync_copy(dst_hbm.at[w], dst_v)

    def step(j, _):
        pltpu.async_copy(ones_v, acc.at[dst_v.at[j]], sem, add=True)
        return 0
    lax.fori_loop(0, NCHUNK, step, 0)

    def drain(j, _):
        pltpu.make_async_copy(ones_v, acc.at[dst_v.at[j]], sem).wait()
        return 0
    lax.fori_loop(0, NCHUNK, drain, 0)
    plsc.subcore_barrier()
    for t in range(ROWS_PER_TILE // CHUNK):
        pltpu.sync_copy(acc.at[pl.ds(base + t * CHUNK, CHUNK)], zbuf_v)
        pltpu.sync_copy(zbuf_v, out_hbm.at[c, pl.ds(base + t * CHUNK, CHUNK)])


def _make_sc_agg(feat):
    @functools.partial(
        pl.kernel,
        out_type=jax.ShapeDtypeStruct((2, N_PAD, feat), jnp.float32),
        mesh=_MESH,
        compiler_params=_SC_PARAMS,
        scratch_types=[
            pltpu.VMEM((NCHUNK, CHUNK), jnp.int32),
            pltpu.VMEM((NCHUNK, CHUNK), jnp.int32),
            pltpu.VMEM((RING * CHUNK, feat), jnp.float32),
            pltpu.VMEM_SHARED((N_PAD, feat), jnp.float32),
        ] + [pltpu.SemaphoreType.DMA] * (2 * RING),
    )
    def agg(hs_hbm, src_hbm, dst_hbm, out_hbm, src_v, dst_v, bufs, acc, *sems):
        gsems, ssems = sems[:RING], sems[RING:]
        c = lax.axis_index("c")
        s = lax.axis_index("s")
        w = s * 2 + c

        def buf(b):
            return bufs.at[pl.ds(b * CHUNK, CHUNK)]

        _fill(bufs, CHUNK, feat, 0.0)
        base = s * ROWS_PER_TILE
        for t in range(ROWS_PER_TILE // CHUNK):
            pltpu.sync_copy(buf(0), acc.at[pl.ds(base + t * CHUNK, CHUNK)])
        plsc.subcore_barrier()
        pltpu.sync_copy(src_hbm.at[w], src_v)
        pltpu.sync_copy(dst_hbm.at[w], dst_v)

        def gather(j, b):
            pltpu.async_copy(hs_hbm.at[src_v.at[j]], buf(b), gsems[b])

        def gwait(j, b):
            pltpu.make_async_copy(hs_hbm.at[src_v.at[j]], buf(b), gsems[b]).wait()

        def scatter(j, b):
            pltpu.async_copy(buf(b), acc.at[dst_v.at[j]], ssems[b], add=True)

        def swait(j, b):
            pltpu.make_async_copy(buf(b), acc.at[dst_v.at[j]], ssems[b]).wait()

        def step(j, k, prefetch, wait_prior):
            if wait_prior:
                swait(j - 2, (k + 2) % RING)
            if prefetch:
                gather(j + 2, (k + 2) % RING)
            gwait(j, k)
            scatter(j, k)

        gather(0, 0)
        gather(1, 1)
        step(0, 0, True, False)
        step(1, 1, True, False)

        def group(t, _):
            j = 4 * t + 2
            for k in range(4):
                step(j + k, (2 + k) % RING, True, True)
            return 0
        lax.fori_loop(0, (NCHUNK - 5) // 4, group, 0)

        for j in range(NCHUNK - 3, NCHUNK):
            step(j, j % RING, j + 2 < NCHUNK, True)
        swait(NCHUNK - 2, (NCHUNK - 2) % RING)
        swait(NCHUNK - 1, (NCHUNK - 1) % RING)
        plsc.subcore_barrier()
        for t in range(ROWS_PER_TILE // CHUNK):
            pltpu.sync_copy(acc.at[pl.ds(base + t * CHUNK, CHUNK)], buf(0))
            pltpu.sync_copy(buf(0), out_hbm.at[c, pl.ds(base + t * CHUNK, CHUNK)])
    return agg


_sc_agg64 = _make_sc_agg(H)
_sc_agg48 = _make_sc_agg(C_PAD)


def _tc1_body(deg_ref, x_ref, w1_ref, hs1_ref, dinv_ref):
    degs = deg_ref[0] + deg_ref[1]
    deg = degs[:, 0:1]
    dinv = jnp.where(deg > 0, lax.rsqrt(deg), 0.0)
    h1 = jnp.dot(x_ref[...], w1_ref[...], preferred_element_type=jnp.float32)
    hs1_ref[...] = h1 * dinv
    dinv_ref[...] = dinv


def _tc2_body(p_ref, dinv_ref, b1_ref, w2_ref, hs2_ref):
    dinv = dinv_ref[...]
    z = (p_ref[0] + p_ref[1]) * dinv + b1_ref[...]
    r = jnp.maximum(z, 0.0)
    h2 = jnp.dot(r, w2_ref[...], preferred_element_type=jnp.float32)
    hs2_ref[...] = h2 * dinv


def _tc3_body(p_ref, dinv_ref, b2_ref, lp_ref, z_ref):
    z = (p_ref[0] + p_ref[1]) * dinv_ref[...] + b2_ref[...]
    col = lax.broadcasted_iota(jnp.int32, (RB, C_PAD), 1)
    valid = col < C
    zm = jnp.where(valid, z, -jnp.inf)
    m = jnp.max(zm, axis=1, keepdims=True)
    e = jnp.where(valid, jnp.exp(z - m), 0.0)
    ssum = jnp.sum(e, axis=1, keepdims=True)
    lp_ref[...] = z - m - jnp.log(ssum)
    z_ref[...] = z


_tc1 = pl.pallas_call(
    _tc1_body,
    grid=(GRID,),
    in_specs=[
        pl.BlockSpec((2, RB, 16), lambda i: (0, i, 0)),
        pl.BlockSpec((RB, D), lambda i: (i, 0)),
        pl.BlockSpec((D, H), lambda i: (0, 0)),
    ],
    out_specs=[
        pl.BlockSpec((RB, H), lambda i: (i, 0)),
        pl.BlockSpec((RB, 1), lambda i: (i, 0)),
    ],
    out_shape=[
        jax.ShapeDtypeStruct((N_PAD, H), jnp.float32),
        jax.ShapeDtypeStruct((N_PAD, 1), jnp.float32),
    ],
)

_tc2 = pl.pallas_call(
    _tc2_body,
    grid=(GRID,),
    in_specs=[
        pl.BlockSpec((2, RB, H), lambda i: (0, i, 0)),
        pl.BlockSpec((RB, 1), lambda i: (i, 0)),
        pl.BlockSpec((1, H), lambda i: (0, 0)),
        pl.BlockSpec((H, C_PAD), lambda i: (0, 0)),
    ],
    out_specs=pl.BlockSpec((RB, C_PAD), lambda i: (i, 0)),
    out_shape=jax.ShapeDtypeStruct((N_PAD, C_PAD), jnp.float32),
)

_tc3 = pl.pallas_call(
    _tc3_body,
    grid=(GRID,),
    in_specs=[
        pl.BlockSpec((2, RB, C_PAD), lambda i: (0, i, 0)),
        pl.BlockSpec((RB, 1), lambda i: (i, 0)),
        pl.BlockSpec((1, C_PAD), lambda i: (0, 0)),
    ],
    out_specs=[
        pl.BlockSpec((RB, C_PAD), lambda i: (i, 0)),
        pl.BlockSpec((RB, C_PAD), lambda i: (i, 0)),
    ],
    out_shape=[
        jax.ShapeDtypeStruct((N_PAD, C_PAD), jnp.float32),
        jax.ShapeDtypeStruct((N_PAD, C_PAD), jnp.float32),
    ],
)


def kernel(x, edge_index, W1, b1, W2, b2):
    loop = jnp.arange(N, dtype=jnp.int32)
    fill = jnp.full((E_PAD - E - N,), N, dtype=jnp.int32)
    src = jnp.concatenate([edge_index[0], loop, fill]).reshape(NW, NCHUNK, CHUNK)
    dst = jnp.concatenate([edge_index[1], loop, fill]).reshape(NW, NCHUNK, CHUNK)

    xp = jnp.pad(x, ((0, N_PAD - N), (0, 0)))
    w2p = jnp.pad(W2, ((0, 0), (0, C_PAD - C)))
    b1r = b1.reshape(1, H)
    b2r = jnp.pad(b2, (0, C_PAD - C)).reshape(1, C_PAD)

    deg_parts = _sc_degree(dst)
    hs1, dinv = _tc1(deg_parts, xp, W1)
    agg1 = _sc_agg64(hs1, src, dst)
    hs2 = _tc2(agg1, dinv, b1r, w2p)
    agg2 = _sc_agg48(hs2, src, dst)
    lp, z = _tc3(agg2, dinv, b2r)

    return (lp[:N, :C], z[:N, :C], jnp.float32(0.0))


# speedup vs baseline: 32.3979x; 1.1842x over previous
import functools

import jax
import jax.numpy as jnp
from jax import lax
from jax.experimental import pallas as pl
from jax.experimental.pallas import tpu as pltpu
from jax.experimental.pallas import tpu_sc as plsc

N = 10000
E = 320000
D = 128
H = 64
C = 40
C_PAD = 48

N_PAD = 10240
ROWS_PER_TILE = 640
NW = 32
CHUNK = 128
NCHUNK = 81
RING = 4
E_PAD = NW * NCHUNK * CHUNK

RB = 256
GRID = N_PAD // RB

_MESH = plsc.VectorSubcoreMesh(core_axis_name="c", subcore_axis_name="s")
_SC_PARAMS = pltpu.CompilerParams(use_tc_tiling_on_sc=False)


def _fill(buf, rows, width, value):
    def body(i, _):
        for k in range(width // 16):
            buf[i, pl.ds(16 * k, 16)] = jnp.full((16,), value, jnp.float32)
        return 0
    lax.fori_loop(0, rows, body, 0)


@functools.partial(
    pl.kernel,
    out_type=jax.ShapeDtypeStruct((2, N_PAD, 16), jnp.float32),
    mesh=_MESH,
    compiler_params=_SC_PARAMS,
    scratch_types=[
        pltpu.VMEM((NCHUNK, CHUNK), jnp.int32),
        pltpu.VMEM((CHUNK, 16), jnp.float32),
        pltpu.VMEM((CHUNK, 16), jnp.float32),
        pltpu.VMEM_SHARED((N_PAD, 16), jnp.float32),
        pltpu.SemaphoreType.DMA,
    ],
)
def _sc_degree(dst_hbm, out_hbm, dst_v, ones_v, zbuf_v, acc, sem):
    c = lax.axis_index("c")
    s = lax.axis_index("s")
    w = s * 2 + c
    _fill(ones_v, CHUNK, 16, 1.0)
    _fill(zbuf_v, CHUNK, 16, 0.0)
    base = s * ROWS_PER_TILE
    for t in range(ROWS_PER_TILE // CHUNK):
        pltpu.sync_copy(zbuf_v, acc.at[pl.ds(base + t * CHUNK, CHUNK)])
    plsc.subcore_barrier()
    pltpu.sync_copy(dst_hbm.at[w], dst_v)

    def step(j, _):
        pltpu.async_copy(ones_v, acc.at[dst_v.at[j]], sem, add=True)
        return 0
    lax.fori_loop(0, NCHUNK, step, 0)

    def drain(j, _):
        pltpu.make_async_copy(ones_v, acc.at[dst_v.at[j]], sem).wait()
        return 0
    lax.fori_loop(0, NCHUNK, drain, 0)
    plsc.subcore_barrier()
    for t in range(ROWS_PER_TILE // CHUNK):
        pltpu.sync_copy(acc.at[pl.ds(base + t * CHUNK, CHUNK)], zbuf_v)
        pltpu.sync_copy(zbuf_v, out_hbm.at[c, pl.ds(base + t * CHUNK, CHUNK)])


def _make_sc_agg(feat):
    @functools.partial(
        pl.kernel,
        out_type=jax.ShapeDtypeStruct((2, N_PAD, feat), jnp.float32),
        mesh=_MESH,
        compiler_params=_SC_PARAMS,
        scratch_types=[
            pltpu.VMEM((NCHUNK, CHUNK), jnp.int32),
            pltpu.VMEM((NCHUNK, CHUNK), jnp.int32),
            pltpu.VMEM((RING * CHUNK, feat), jnp.float32),
            pltpu.VMEM_SHARED((N_PAD, feat), jnp.float32),
        ] + [pltpu.SemaphoreType.DMA] * (2 * RING),
    )
    def agg(hs_hbm, src_hbm, dst_hbm, out_hbm, src_v, dst_v, bufs, acc, *sems):
        gsems, ssems = sems[:RING], sems[RING:]
        c = lax.axis_index("c")
        s = lax.axis_index("s")
        w = s * 2 + c

        def buf(b):
            return bufs.at[pl.ds(b * CHUNK, CHUNK)]

        _fill(bufs, CHUNK, feat, 0.0)
        base = s * ROWS_PER_TILE
        for t in range(ROWS_PER_TILE // CHUNK):
            pltpu.sync_copy(buf(0), acc.at[pl.ds(base + t * CHUNK, CHUNK)])
        plsc.subcore_barrier()
        pltpu.sync_copy(src_hbm.at[w], src_v)
        pltpu.sync_copy(dst_hbm.at[w], dst_v)

        def gather(j, b):
            pltpu.async_copy(hs_hbm.at[src_v.at[j]], buf(b), gsems[b])

        def gwait(j, b):
            pltpu.make_async_copy(hs_hbm.at[src_v.at[j]], buf(b), gsems[b]).wait()

        def scatter(j, b):
            pltpu.async_copy(buf(b), acc.at[dst_v.at[j]], ssems[b], add=True)

        def swait(j, b):
            pltpu.make_async_copy(buf(b), acc.at[dst_v.at[j]], ssems[b]).wait()

        def step(j, k, prefetch, wait_prior):
            if wait_prior:
                swait(j - 2, (k + 2) % RING)
            if prefetch:
                gather(j + 2, (k + 2) % RING)
            gwait(j, k)
            scatter(j, k)

        gather(0, 0)
        gather(1, 1)
        step(0, 0, True, False)
        step(1, 1, True, False)

        def group(t, _):
            j = 4 * t + 2
            for k in range(4):
                step(j + k, (2 + k) % RING, True, True)
            return 0
        lax.fori_loop(0, (NCHUNK - 5) // 4, group, 0)

        for j in range(NCHUNK - 3, NCHUNK):
            step(j, j % RING, j + 2 < NCHUNK, True)
        swait(NCHUNK - 2, (NCHUNK - 2) % RING)
        swait(NCHUNK - 1, (NCHUNK - 1) % RING)
        plsc.subcore_barrier()
        for t in range(ROWS_PER_TILE // CHUNK):
            pltpu.sync_copy(acc.at[pl.ds(base + t * CHUNK, CHUNK)], buf(0))
            pltpu.sync_copy(buf(0), out_hbm.at[c, pl.ds(base + t * CHUNK, CHUNK)])
    return agg


_sc_agg64 = _make_sc_agg(H)
_sc_agg48 = _make_sc_agg(C_PAD)


def _tc1a_body(x_ref, w1_ref, h1_ref):
    h1_ref[...] = jnp.dot(x_ref[...], w1_ref[...],
                          preferred_element_type=jnp.float32)


def _tc1b_body(deg_ref, h1_ref, hs1_ref, dinv_ref):
    degs = deg_ref[0] + deg_ref[1]
    deg = degs[:, 0:1]
    dinv = jnp.where(deg > 0, lax.rsqrt(deg), 0.0)
    hs1_ref[...] = h1_ref[...] * dinv
    dinv_ref[...] = dinv


def _tc2_body(p_ref, dinv_ref, b1_ref, w2_ref, hs2_ref):
    dinv = dinv_ref[...]
    z = (p_ref[0] + p_ref[1]) * dinv + b1_ref[...]
    r = jnp.maximum(z, 0.0)
    h2 = jnp.dot(r, w2_ref[...], preferred_element_type=jnp.float32)
    hs2_ref[...] = h2 * dinv


def _tc3_body(p_ref, dinv_ref, b2_ref, lp_ref, z_ref):
    z = (p_ref[0] + p_ref[1]) * dinv_ref[...] + b2_ref[...]
    col = lax.broadcasted_iota(jnp.int32, (RB, C_PAD), 1)
    valid = col < C
    zm = jnp.where(valid, z, -jnp.inf)
    m = jnp.max(zm, axis=1, keepdims=True)
    e = jnp.where(valid, jnp.exp(z - m), 0.0)
    ssum = jnp.sum(e, axis=1, keepdims=True)
    lp_ref[...] = z - m - jnp.log(ssum)
    z_ref[...] = z


_tc1a = pl.pallas_call(
    _tc1a_body,
    grid=(GRID,),
    in_specs=[
        pl.BlockSpec((RB, D), lambda i: (i, 0)),
        pl.BlockSpec((D, H), lambda i: (0, 0)),
    ],
    out_specs=pl.BlockSpec((RB, H), lambda i: (i, 0)),
    out_shape=jax.ShapeDtypeStruct((N_PAD, H), jnp.float32),
)

_tc1b = pl.pallas_call(
    _tc1b_body,
    grid=(GRID,),
    in_specs=[
        pl.BlockSpec((2, RB, 16), lambda i: (0, i, 0)),
        pl.BlockSpec((RB, H), lambda i: (i, 0)),
    ],
    out_specs=[
        pl.BlockSpec((RB, H), lambda i: (i, 0)),
        pl.BlockSpec((RB, 1), lambda i: (i, 0)),
    ],
    out_shape=[
        jax.ShapeDtypeStruct((N_PAD, H), jnp.float32),
        jax.ShapeDtypeStruct((N_PAD, 1), jnp.float32),
    ],
)

_tc2 = pl.pallas_call(
    _tc2_body,
    grid=(GRID,),
    in_specs=[
        pl.BlockSpec((2, RB, H), lambda i: (0, i, 0)),
        pl.BlockSpec((RB, 1), lambda i: (i, 0)),
        pl.BlockSpec((1, H), lambda i: (0, 0)),
        pl.BlockSpec((H, C_PAD), lambda i: (0, 0)),
    ],
    out_specs=pl.BlockSpec((RB, C_PAD), lambda i: (i, 0)),
    out_shape=jax.ShapeDtypeStruct((N_PAD, C_PAD), jnp.float32),
)

_tc3 = pl.pallas_call(
    _tc3_body,
    grid=(GRID,),
    in_specs=[
        pl.BlockSpec((2, RB, C_PAD), lambda i: (0, i, 0)),
        pl.BlockSpec((RB, 1), lambda i: (i, 0)),
        pl.BlockSpec((1, C_PAD), lambda i: (0, 0)),
    ],
    out_specs=[
        pl.BlockSpec((RB, C_PAD), lambda i: (i, 0)),
        pl.BlockSpec((RB, C_PAD), lambda i: (i, 0)),
    ],
    out_shape=[
        jax.ShapeDtypeStruct((N_PAD, C_PAD), jnp.float32),
        jax.ShapeDtypeStruct((N_PAD, C_PAD), jnp.float32),
    ],
)


def kernel(x, edge_index, W1, b1, W2, b2):
    loop = jnp.arange(N, dtype=jnp.int32)
    fill = N + jnp.arange(E_PAD - E - N, dtype=jnp.int32) % (N_PAD - N)
    src = jnp.concatenate([edge_index[0], loop, fill]).reshape(NW, NCHUNK, CHUNK)
    dst = jnp.concatenate([edge_index[1], loop, fill]).reshape(NW, NCHUNK, CHUNK)

    xp = jnp.pad(x, ((0, N_PAD - N), (0, 0)))
    w2p = jnp.pad(W2, ((0, 0), (0, C_PAD - C)))
    b1r = b1.reshape(1, H)
    b2r = jnp.pad(b2, (0, C_PAD - C)).reshape(1, C_PAD)

    deg_parts = _sc_degree(dst)
    h1 = _tc1a(xp, W1)
    hs1, dinv = _tc1b(deg_parts, h1)
    agg1 = _sc_agg64(hs1, src, dst)
    hs2 = _tc2(agg1, dinv, b1r, w2p)
    agg2 = _sc_agg48(hs2, src, dst)
    lp, z = _tc3(agg2, dinv, b2r)

    return (lp[:N, :C], z[:N, :C], jnp.float32(0.0))
